# Initial kernel scaffold; baseline (speedup 1.0000x reference)
import jax
import jax.numpy as jnp
from jax.experimental import pallas as pl


def kernel(x, edge_index, W1, b1, a1, W2, b2, a2):
    raise NotImplementedError("write your pallas kernel here")


# speedup vs baseline: 10.4102x; 10.4102x over previous
import functools

import jax
import jax.numpy as jnp
from jax import lax
from jax.experimental import pallas as pl
from jax.experimental.pallas import tpu as pltpu
from jax.experimental.pallas import tpu_sc as plsc

NC = 2
NS = 16
L = 16
NW = NC * NS
CH = 128


def _sc_degree(n_pad, pw):
    mesh = plsc.VectorSubcoreMesh(core_axis_name="c", subcore_axis_name="s")

    @functools.partial(
        pl.kernel,
        out_type=jax.ShapeDtypeStruct((NC, NS, n_pad), jnp.float32),
        mesh=mesh,
        scratch_types=[
            pltpu.VMEM((pw,), jnp.int32),
            pltpu.VMEM((n_pad,), jnp.float32),
        ],
        compiler_params=pltpu.CompilerParams(needs_layout_passes=False),
    )
    def deg_kernel(dst_hbm, out_hbm, idx_v, deg_v):
        c = lax.axis_index("c")
        s = lax.axis_index("s")
        pltpu.sync_copy(dst_hbm.at[c, s], idx_v)
        zeros16 = jnp.zeros((L,), jnp.float32)

        def zb(i, carry):
            deg_v[pl.ds(i * L, L)] = zeros16
            return carry

        lax.fori_loop(0, n_pad // L, zb, 0)
        ones16 = jnp.ones((L,), jnp.float32)

        def body(i, carry):
            idx = idx_v[pl.ds(i * L, L)]
            plsc.addupdate_scatter(deg_v, [idx], ones16)
            return carry

        lax.fori_loop(0, pw // L, body, 0)
        pltpu.sync_copy(deg_v, out_hbm.at[c, s])

    return deg_kernel


def _sc_scatter(n_pad, jw, d):
    mesh = plsc.VectorSubcoreMesh(core_axis_name="c", subcore_axis_name="s")
    zr = 64
    rt = n_pad // NS

    @functools.partial(
        pl.kernel,
        out_type=jax.ShapeDtypeStruct((NC, n_pad, d), jnp.float32),
        mesh=mesh,
        scratch_types=[
            pltpu.VMEM((jw, CH), jnp.int32),
            pltpu.VMEM((jw, CH), jnp.int32),
            pltpu.VMEM((CH, d), jnp.float32),
            pltpu.VMEM((zr, d), jnp.float32),
            pltpu.VMEM_SHARED((n_pad, d), jnp.float32),
            pltpu.SemaphoreType.DMA,
        ],
    )
    def scat_kernel(hp_hbm, src_hbm, dst_hbm, out_hbm,
                    src_v, dst_v, rows_v, zero_v, acc_sh, gsem):
        c = lax.axis_index("c")
        s = lax.axis_index("s")
        pltpu.sync_copy(src_hbm.at[c, s], src_v)
        pltpu.sync_copy(dst_hbm.at[c, s], dst_v)

        zeros16 = jnp.zeros((L,), jnp.float32)

        def zb(i, carry):
            for k in range(d // L):
                zero_v[i, pl.ds(k * L, L)] = zeros16
            return carry

        lax.fori_loop(0, zr, zb, 0)

        def zb2(i, carry):
            pltpu.sync_copy(zero_v, acc_sh.at[pl.ds(s * rt + i * zr, zr)])
            return carry

        lax.fori_loop(0, rt // zr, zb2, 0)
        plsc.subcore_barrier()

        def body(j, carry):
            pltpu.async_copy(hp_hbm.at[src_v.at[j]], rows_v, gsem).wait()
            pltpu.sync_copy(rows_v, acc_sh.at[dst_v.at[j]], add=True)
            return carry

        lax.fori_loop(0, jw, body, 0)
        plsc.subcore_barrier()
        pltpu.sync_copy(acc_sh.at[pl.ds(s * rt, rt)],
                        out_hbm.at[c, pl.ds(s * rt, rt)])

    return scat_kernel


def _tc_first(deg_p, x, w, blk):
    n, d = x.shape

    def body(dp_ref, x_ref, w_ref, o_ref):
        deg = jnp.sum(dp_ref[...], axis=(0, 1)) + 1.0
        dinv = lax.rsqrt(deg)
        h = jnp.dot(x_ref[...], w_ref[...], preferred_element_type=jnp.float32)
        o_ref[...] = h * dinv

    return pl.pallas_call(
        body,
        grid=(n // blk,),
        in_specs=[
            pl.BlockSpec((NC, NS, blk, 1), lambda i: (0, 0, i, 0)),
            pl.BlockSpec((blk, d), lambda i: (i, 0)),
            pl.BlockSpec((d, d), lambda i: (0, 0)),
        ],
        out_specs=pl.BlockSpec((blk, d), lambda i: (i, 0)),
        out_shape=jax.ShapeDtypeStruct((n, d), jnp.float32),
    )(deg_p, x, w)


def _tc_mid(deg_p, p, hp, b, a, w, blk):
    n, d = hp.shape

    def body(dp_ref, p_ref, hp_ref, b_ref, a_ref, w_ref, o_ref):
        deg = jnp.sum(dp_ref[...], axis=(0, 1)) + 1.0
        dinv = lax.rsqrt(deg)
        t = dinv * (p_ref[0] + p_ref[1] + hp_ref[...]) + b_ref[...]
        u = jnp.where(t >= 0, t, a_ref[...] * t)
        h = jnp.dot(u, w_ref[...], preferred_element_type=jnp.float32)
        o_ref[...] = h * dinv

    return pl.pallas_call(
        body,
        grid=(n // blk,),
        in_specs=[
            pl.BlockSpec((NC, NS, blk, 1), lambda i: (0, 0, i, 0)),
            pl.BlockSpec((NC, blk, d), lambda i: (0, i, 0)),
            pl.BlockSpec((blk, d), lambda i: (i, 0)),
            pl.BlockSpec((1, d), lambda i: (0, 0)),
            pl.BlockSpec((1, 1), lambda i: (0, 0)),
            pl.BlockSpec((d, d), lambda i: (0, 0)),
        ],
        out_specs=pl.BlockSpec((blk, d), lambda i: (i, 0)),
        out_shape=jax.ShapeDtypeStruct((n, d), jnp.float32),
    )(deg_p, p, hp, b, a, w)


def _tc_last(deg_p, p, hp, b, a, blk):
    n, d = hp.shape

    def body(dp_ref, p_ref, hp_ref, b_ref, a_ref, o_ref):
        deg = jnp.sum(dp_ref[...], axis=(0, 1)) + 1.0
        dinv = lax.rsqrt(deg)
        t = dinv * (p_ref[0] + p_ref[1] + hp_ref[...]) + b_ref[...]
        o_ref[...] = jnp.where(t >= 0, t, a_ref[...] * t)

    return pl.pallas_call(
        body,
        grid=(n // blk,),
        in_specs=[
            pl.BlockSpec((NC, NS, blk, 1), lambda i: (0, 0, i, 0)),
            pl.BlockSpec((NC, blk, d), lambda i: (0, i, 0)),
            pl.BlockSpec((blk, d), lambda i: (i, 0)),
            pl.BlockSpec((1, d), lambda i: (0, 0)),
            pl.BlockSpec((1, 1), lambda i: (0, 0)),
        ],
        out_specs=pl.BlockSpec((blk, d), lambda i: (i, 0)),
        out_shape=jax.ShapeDtypeStruct((n, d), jnp.float32),
    )(deg_p, p, hp, b, a)


def kernel(x, edge_index, W1, b1, a1, W2, b2, a2):
    n, d = x.shape
    e = edge_index.shape[1]
    src = edge_index[0].astype(jnp.int32)
    dst = edge_index[1].astype(jnp.int32)

    ep = -(-e // (NW * CH)) * (NW * CH)
    jw = ep // (NW * CH)
    pw = jw * CH
    n_pad = -(-(n + 1) // 2048) * 2048

    pad = ep - e
    srcp = jnp.concatenate([src, jnp.zeros((pad,), jnp.int32)])
    dstp = jnp.concatenate([dst, jnp.full((pad,), n, jnp.int32)])
    src_ch = srcp.reshape(NC, NS, jw, CH)
    dst_ch = dstp.reshape(NC, NS, jw, CH)
    dst_flat = dstp.reshape(NC, NS, pw)

    blk = 1000 if n % 1000 == 0 else 8
    assert n % blk == 0

    deg_p = _sc_degree(n_pad, pw)(dst_flat)[:, :, :, None]

    scat = _sc_scatter(n_pad, jw, d)
    hp1 = _tc_first(deg_p, x, W1, blk)
    p1 = scat(hp1, src_ch, dst_ch)
    hp2 = _tc_mid(deg_p, p1, hp1, b1.reshape(1, d), a1.reshape(1, 1), W2, blk)
    p2 = scat(hp2, src_ch, dst_ch)
    return _tc_last(deg_p, p2, hp2, b2.reshape(1, d), a2.reshape(1, 1), blk)


# speedup vs baseline: 11.3197x; 1.0874x over previous
import functools

import jax
import jax.numpy as jnp
from jax import lax
from jax.experimental import pallas as pl
from jax.experimental.pallas import tpu as pltpu
from jax.experimental.pallas import tpu_sc as plsc

NC = 2
NS = 16
L = 16
NW = NC * NS
CH = 128


def _sc_degree(n_pad, pw):
    mesh = plsc.VectorSubcoreMesh(core_axis_name="c", subcore_axis_name="s")

    dc = 1024

    @functools.partial(
        pl.kernel,
        out_type=jax.ShapeDtypeStruct((NC, NS, n_pad), jnp.float32),
        mesh=mesh,
        scratch_types=[
            pltpu.VMEM((dc,), jnp.int32),
            pltpu.VMEM((n_pad,), jnp.float32),
        ],
        compiler_params=pltpu.CompilerParams(needs_layout_passes=False),
    )
    def deg_kernel(dst_hbm, out_hbm, idx_v, deg_v):
        c = lax.axis_index("c")
        s = lax.axis_index("s")
        zeros16 = jnp.zeros((L,), jnp.float32)

        def zb(i, carry):
            deg_v[pl.ds(i * L, L)] = zeros16
            return carry

        lax.fori_loop(0, n_pad // L, zb, 0)
        ones16 = jnp.ones((L,), jnp.float32)

        def body(i, carry):
            idx = idx_v[pl.ds(i * L, L)]
            plsc.addupdate_scatter(deg_v, [idx], ones16)
            return carry

        for g in range(0, pw, dc):
            sz = min(dc, pw - g)
            pltpu.sync_copy(dst_hbm.at[c, s, pl.ds(g, sz)],
                            idx_v.at[pl.ds(0, sz)])
            lax.fori_loop(0, sz // L, body, 0)
        pltpu.sync_copy(deg_v, out_hbm.at[c, s])

    return deg_kernel


def _sc_scatter(n_pad, jw, d):
    mesh = plsc.VectorSubcoreMesh(core_axis_name="c", subcore_axis_name="s")
    rt = n_pad // NS

    @functools.partial(
        pl.kernel,
        out_type=jax.ShapeDtypeStruct((NC, n_pad, d), jnp.float32),
        mesh=mesh,
        scratch_types=[
            pltpu.VMEM((jw, CH), jnp.int32),
            pltpu.VMEM((2, CH), jnp.int32),
            pltpu.VMEM((2, CH), jnp.int32),
            pltpu.VMEM((2, CH, d), jnp.float32),
            pltpu.VMEM_SHARED((n_pad, d), jnp.float32),
            pltpu.SemaphoreType.DMA,
            pltpu.SemaphoreType.DMA,
        ],
    )
    def scat_kernel(hp_hbm, pk_hbm, out_hbm,
                    pk_v, su_v, du_v, rows_v, acc_sh, gsem, ssem):
        c = lax.axis_index("c")
        s = lax.axis_index("s")
        pltpu.sync_copy(pk_hbm.at[c, s], pk_v)

        zeros16 = jnp.zeros((L,), jnp.float32)

        def zb(i, carry):
            for k in range(d // L):
                rows_v[1, i, pl.ds(k * L, L)] = zeros16
            return carry

        lax.fori_loop(0, CH, zb, 0)

        def zb2(i, carry):
            pltpu.sync_copy(rows_v.at[1], acc_sh.at[pl.ds(s * rt + i * CH, CH)])
            return carry

        lax.fori_loop(0, rt // CH, zb2, 0)
        if rt % CH:
            pltpu.sync_copy(
                rows_v.at[1, pl.ds(0, rt % CH)],
                acc_sh.at[pl.ds(s * rt + (rt // CH) * CH, rt % CH)])
        plsc.subcore_barrier()

        mask16 = jnp.full((L,), 0xFFFF, jnp.int32)

        def unpack(j, b):
            def ub(i, carry):
                p = pk_v[j, pl.ds(i * L, L)]
                su_v[b, pl.ds(i * L, L)] = jnp.bitwise_and(p, mask16)
                du_v[b, pl.ds(i * L, L)] = lax.shift_right_logical(p, 16)
                return carry

            lax.fori_loop(0, CH // L, ub, 0)

        unpack(0, 0)
        pltpu.async_copy(hp_hbm.at[su_v.at[0]], rows_v.at[0], gsem)

        def body(j, carry):
            b = lax.rem(j, 2)
            pltpu.make_async_copy(hp_hbm.at[su_v.at[b]],
                                  rows_v.at[b], gsem).wait()
            pltpu.async_copy(rows_v.at[b], acc_sh.at[du_v.at[b]], ssem,
                             add=True)

            @pl.when(j >= 1)
            def _():
                pltpu.make_async_copy(rows_v.at[1 - b],
                                      acc_sh.at[du_v.at[1 - b]], ssem).wait()

            @pl.when(j + 1 < jw)
            def _():
                unpack(j + 1, 1 - b)
                pltpu.async_copy(hp_hbm.at[su_v.at[1 - b]],
                                 rows_v.at[1 - b], gsem)

            return carry

        lax.fori_loop(0, jw, body, 0)
        pltpu.make_async_copy(rows_v.at[(jw - 1) % 2],
                              acc_sh.at[du_v.at[(jw - 1) % 2]], ssem).wait()
        plsc.subcore_barrier()
        pltpu.sync_copy(acc_sh.at[pl.ds(s * rt, rt)],
                        out_hbm.at[c, pl.ds(s * rt, rt)])

    return scat_kernel


def _tc_first(deg_p, x, w, blk):
    n, d = x.shape

    def body(dp_ref, x_ref, w_ref, o_ref):
        deg = jnp.sum(dp_ref[...], axis=(0, 1)) + 1.0
        dinv = lax.rsqrt(deg)
        h = jnp.dot(x_ref[...], w_ref[...], preferred_element_type=jnp.float32)
        o_ref[...] = h * dinv

    return pl.pallas_call(
        body,
        grid=(n // blk,),
        in_specs=[
            pl.BlockSpec((NC, NS, blk, 1), lambda i: (0, 0, i, 0)),
            pl.BlockSpec((blk, d), lambda i: (i, 0)),
            pl.BlockSpec((d, d), lambda i: (0, 0)),
        ],
        out_specs=pl.BlockSpec((blk, d), lambda i: (i, 0)),
        out_shape=jax.ShapeDtypeStruct((n, d), jnp.float32),
    )(deg_p, x, w)


def _tc_mid(deg_p, p, hp, b, a, w, blk):
    n, d = hp.shape

    def body(dp_ref, p_ref, hp_ref, b_ref, a_ref, w_ref, o_ref):
        deg = jnp.sum(dp_ref[...], axis=(0, 1)) + 1.0
        dinv = lax.rsqrt(deg)
        t = dinv * (p_ref[0] + p_ref[1] + hp_ref[...]) + b_ref[...]
        u = jnp.where(t >= 0, t, a_ref[...] * t)
        h = jnp.dot(u, w_ref[...], preferred_element_type=jnp.float32)
        o_ref[...] = h * dinv

    return pl.pallas_call(
        body,
        grid=(n // blk,),
        in_specs=[
            pl.BlockSpec((NC, NS, blk, 1), lambda i: (0, 0, i, 0)),
            pl.BlockSpec((NC, blk, d), lambda i: (0, i, 0)),
            pl.BlockSpec((blk, d), lambda i: (i, 0)),
            pl.BlockSpec((1, d), lambda i: (0, 0)),
            pl.BlockSpec((1, 1), lambda i: (0, 0)),
            pl.BlockSpec((d, d), lambda i: (0, 0)),
        ],
        out_specs=pl.BlockSpec((blk, d), lambda i: (i, 0)),
        out_shape=jax.ShapeDtypeStruct((n, d), jnp.float32),
    )(deg_p, p, hp, b, a, w)


def _tc_last(deg_p, p, hp, b, a, blk):
    n, d = hp.shape

    def body(dp_ref, p_ref, hp_ref, b_ref, a_ref, o_ref):
        deg = jnp.sum(dp_ref[...], axis=(0, 1)) + 1.0
        dinv = lax.rsqrt(deg)
        t = dinv * (p_ref[0] + p_ref[1] + hp_ref[...]) + b_ref[...]
        o_ref[...] = jnp.where(t >= 0, t, a_ref[...] * t)

    return pl.pallas_call(
        body,
        grid=(n // blk,),
        in_specs=[
            pl.BlockSpec((NC, NS, blk, 1), lambda i: (0, 0, i, 0)),
            pl.BlockSpec((NC, blk, d), lambda i: (0, i, 0)),
            pl.BlockSpec((blk, d), lambda i: (i, 0)),
            pl.BlockSpec((1, d), lambda i: (0, 0)),
            pl.BlockSpec((1, 1), lambda i: (0, 0)),
        ],
        out_specs=pl.BlockSpec((blk, d), lambda i: (i, 0)),
        out_shape=jax.ShapeDtypeStruct((n, d), jnp.float32),
    )(deg_p, p, hp, b, a)


def kernel(x, edge_index, W1, b1, a1, W2, b2, a2):
    n, d = x.shape
    e = edge_index.shape[1]
    src = edge_index[0].astype(jnp.int32)
    dst = edge_index[1].astype(jnp.int32)

    ep = -(-e // (NW * 128)) * (NW * 128)
    pw = ep // NW
    jw = pw // CH
    n_pad = -(-(n + 1) // 128) * 128

    pad = ep - e
    srcp = jnp.concatenate([src, jnp.zeros((pad,), jnp.int32)])
    dstp = jnp.concatenate([dst, jnp.full((pad,), n, jnp.int32)])
    packed = (srcp | (dstp << 16)).reshape(NC, NS, jw, CH)
    dst_flat = dstp.reshape(NC, NS, pw)

    blk = 1000 if n % 1000 == 0 else 8
    assert n % blk == 0

    deg_p = _sc_degree(n_pad, pw)(dst_flat)[:, :, :, None]

    scat = _sc_scatter(n_pad, jw, d)
    hp1 = _tc_first(deg_p, x, W1, blk)
    p1 = scat(hp1, packed)
    hp2 = _tc_mid(deg_p, p1, hp1, b1.reshape(1, d), a1.reshape(1, 1), W2, blk)
    p2 = scat(hp2, packed)
    return _tc_last(deg_p, p2, hp2, b2.reshape(1, d), a2.reshape(1, 1), blk)
